# baseline (speedup 1.0000x reference)
import jax
import jax.numpy as jnp
from jax.experimental import pallas as pl

_Q = 65536
_B = 1024
_D = 128
_BLK = 1024
_N = _Q // _BLK


def _body(new_ref, keys_ref, old_ref, upd_ref):
    i = pl.program_id(0)
    kb = keys_ref[...]
    old_ref[...] = kb

    @pl.when(i < _N - 1)
    def _():
        upd_ref[...] = kb

    @pl.when(i == _N - 1)
    def _():
        upd_ref[...] = new_ref[...]


def kernel(new_keys, keys):
    old, upd = pl.pallas_call(
        _body,
        grid=(_N,),
        in_specs=[
            pl.BlockSpec((_B, _D), lambda i: (0, 0)),
            pl.BlockSpec((_BLK, _D), lambda i: (i, 0)),
        ],
        out_specs=[
            pl.BlockSpec((_BLK, _D), lambda i: (i, 0)),
            pl.BlockSpec((_BLK, _D), lambda i: ((i + 1) % _N, 0)),
        ],
        out_shape=[
            jax.ShapeDtypeStruct((_Q, _D), jnp.float32),
            jax.ShapeDtypeStruct((_Q, _D), jnp.float32),
        ],
    )(new_keys, keys)
    return (old, upd)
